# baseline (speedup 1.0000x reference)
import math

import jax
import jax.numpy as jnp
from jax import lax
from jax.experimental import pallas as pl
from jax.experimental.pallas import tpu as pltpu
from jax.experimental.pallas import tpu_sc as plsc

D_MODEL = 64
SCALE = math.sqrt(D_MODEL)

NC = 2
NS = 16
NW = NC * NS

BATCH = 1024
SEQ = 200
ROWS_PER_W = BATCH // NW
SPLIT = 128
REM = SEQ - SPLIT
NBUF = 4
NROUND = ROWS_PER_W // NBUF


def _emb_kernel(table_hbm, x_hbm, out_hbm, idx_v, *scr):
    gbufs = scr[0:NBUF]
    obufs = scr[NBUF:2 * NBUF]
    gsems = scr[2 * NBUF:3 * NBUF]
    psems = scr[3 * NBUF:4 * NBUF]

    wid = lax.axis_index("s") * NC + lax.axis_index("c")
    row0 = wid * ROWS_PER_W

    pltpu.sync_copy(x_hbm.at[pl.ds(row0, ROWS_PER_W)], idx_v)

    def start_gather(r, b):
        pltpu.async_copy(
            table_hbm.at[idx_v.at[r, pl.ds(0, SPLIT)]],
            gbufs[b].at[pl.ds(0, SPLIT)], gsems[b])
        pltpu.async_copy(
            table_hbm.at[idx_v.at[r, pl.ds(SPLIT, REM)]],
            gbufs[b].at[pl.ds(SPLIT, REM)], gsems[b])

    def wait_gather(b):
        pltpu.make_async_copy(table_hbm.at[idx_v.at[0]], gbufs[b], gsems[b]).wait()

    def start_put(r, b):
        pltpu.async_copy(obufs[b], out_hbm.at[row0 + r], psems[b])

    def wait_put(b):
        pltpu.make_async_copy(obufs[b], out_hbm.at[row0], psems[b]).wait()

    def mul_row(b):
        gb, ob = gbufs[b], obufs[b]

        def body(t):
            for d in range(4):
                sl = pl.ds(d * 16, 16)
                ob[t, sl] = gb[t, sl] * SCALE

        pl.loop(0, SEQ, unroll=4)(body)

    for b in range(NBUF):
        start_gather(b, b)

    for b in range(NBUF):
        wait_gather(b)
        mul_row(b)
        start_put(b, b)
        start_gather(NBUF + b, b)

    def round_body(g):
        for b in range(NBUF):
            r = g * NBUF + b
            wait_gather(b)
            wait_put(b)
            mul_row(b)
            start_put(r, b)
            start_gather(r + NBUF, b)

    pl.loop(1, NROUND - 1)(round_body)

    for b in range(NBUF):
        r = (NROUND - 1) * NBUF + b
        wait_gather(b)
        wait_put(b)
        mul_row(b)
        start_put(r, b)

    for b in range(NBUF):
        wait_put(b)


@jax.jit
def kernel(x, table):
    mesh = plsc.VectorSubcoreMesh(core_axis_name="c", subcore_axis_name="s")
    run = pl.kernel(
        _emb_kernel,
        out_type=jax.ShapeDtypeStruct((BATCH, SEQ, D_MODEL), jnp.float32),
        mesh=mesh,
        scratch_types=(
            [pltpu.VMEM((ROWS_PER_W, SEQ), jnp.int32)]
            + [pltpu.VMEM((SEQ, D_MODEL), jnp.float32) for _ in range(2 * NBUF)]
            + [pltpu.SemaphoreType.DMA for _ in range(2 * NBUF)]
        ),
        compiler_params=pltpu.CompilerParams(use_tc_tiling_on_sc=False),
    )
    return run(table, x.astype(jnp.int32))


# speedup vs baseline: 1.1003x; 1.1003x over previous
import math

import jax
import jax.numpy as jnp
from jax import lax
from jax.experimental import pallas as pl
from jax.experimental.pallas import tpu as pltpu
from jax.experimental.pallas import tpu_sc as plsc

D_MODEL = 64
SCALE = math.sqrt(D_MODEL)

NC = 2
NS = 16
NW = NC * NS

BATCH = 1024
SEQ = 200
ROWS_PER_W = BATCH // NW
SPLIT = 128
REM = SEQ - SPLIT
NBUF = 4
NROUND = ROWS_PER_W // NBUF


def _emb_kernel(table_hbm, x_hbm, out_hbm, idx_v, *scr):
    gbufs = scr[0:NBUF]
    obufs = scr[NBUF:2 * NBUF]
    gsems = scr[2 * NBUF:3 * NBUF]
    psems = scr[3 * NBUF:4 * NBUF]

    wid = lax.axis_index("s") * NC + lax.axis_index("c")
    row0 = wid * ROWS_PER_W

    pltpu.sync_copy(x_hbm.at[pl.ds(row0, ROWS_PER_W)], idx_v)

    def start_gather(r, b):
        pltpu.async_copy(
            table_hbm.at[idx_v.at[r, pl.ds(0, SPLIT)]],
            gbufs[b].at[pl.ds(0, SPLIT)], gsems[b])
        pltpu.async_copy(
            table_hbm.at[idx_v.at[r, pl.ds(SPLIT, REM)]],
            gbufs[b].at[pl.ds(SPLIT, REM)], gsems[b])

    def wait_gather(b):
        pltpu.make_async_copy(table_hbm.at[idx_v.at[0]], gbufs[b], gsems[b]).wait()

    def start_put(r, b):
        pltpu.async_copy(obufs[b], out_hbm.at[row0 + r, :, pl.ds(0, D_MODEL)], psems[b])

    def wait_put(b):
        pltpu.make_async_copy(obufs[b], out_hbm.at[row0, :, pl.ds(0, D_MODEL)], psems[b]).wait()

    def mul_row(b):
        gb, ob = gbufs[b], obufs[b]

        def body(t):
            for d in range(4):
                sl = pl.ds(d * 16, 16)
                ob[t, sl] = gb[t, sl] * SCALE

        pl.loop(0, SEQ, unroll=4)(body)

    for b in range(NBUF):
        start_gather(b, b)

    for b in range(NBUF):
        wait_gather(b)
        mul_row(b)
        start_put(b, b)
        start_gather(NBUF + b, b)

    def round_body(g):
        for b in range(NBUF):
            r = g * NBUF + b
            wait_gather(b)
            wait_put(b)
            mul_row(b)
            start_put(r, b)
            start_gather(r + NBUF, b)

    pl.loop(1, NROUND - 1)(round_body)

    for b in range(NBUF):
        r = (NROUND - 1) * NBUF + b
        wait_gather(b)
        wait_put(b)
        mul_row(b)
        start_put(r, b)

    for b in range(NBUF):
        wait_put(b)


@jax.jit
def kernel(x, table):
    mesh = plsc.VectorSubcoreMesh(core_axis_name="c", subcore_axis_name="s")
    run = pl.kernel(
        _emb_kernel,
        out_type=jax.ShapeDtypeStruct((BATCH, SEQ, 128), jnp.float32),
        mesh=mesh,
        scratch_types=(
            [pltpu.VMEM((ROWS_PER_W, SEQ), jnp.int32)]
            + [pltpu.VMEM((SEQ, D_MODEL), jnp.float32) for _ in range(2 * NBUF)]
            + [pltpu.SemaphoreType.DMA for _ in range(2 * NBUF)]
        ),
        compiler_params=pltpu.CompilerParams(use_tc_tiling_on_sc=False),
    )
    return run(table, x.astype(jnp.int32))[:, :, :D_MODEL]


# speedup vs baseline: 1.1005x; 1.0002x over previous
import math

import jax
import jax.numpy as jnp
from jax import lax
from jax.experimental import pallas as pl
from jax.experimental.pallas import tpu as pltpu
from jax.experimental.pallas import tpu_sc as plsc

D_MODEL = 64
SCALE = math.sqrt(D_MODEL)

NC = 2
NS = 16
NW = NC * NS

BATCH = 1024
SEQ = 200
ROWS_PER_W = BATCH // NW
SPLIT = 128
REM = SEQ - SPLIT
NBUF = 4
NROUND = ROWS_PER_W // NBUF


def _emb_kernel(table_hbm, x_hbm, out_hbm, idx_v, *scr):
    gbufs = scr[0:NBUF]
    obufs = scr[NBUF:2 * NBUF]
    gsems = scr[2 * NBUF:3 * NBUF]
    psems = scr[3 * NBUF:4 * NBUF]

    wid = lax.axis_index("s") * NC + lax.axis_index("c")
    row0 = wid * ROWS_PER_W

    pltpu.sync_copy(x_hbm.at[pl.ds(row0, ROWS_PER_W)], idx_v)

    def start_gather(r, b):
        pltpu.async_copy(
            table_hbm.at[idx_v.at[r, pl.ds(0, SPLIT)]],
            gbufs[b].at[pl.ds(0, SPLIT)], gsems[b])
        pltpu.async_copy(
            table_hbm.at[idx_v.at[r, pl.ds(SPLIT, REM)]],
            gbufs[b].at[pl.ds(SPLIT, REM)], gsems[b])

    def wait_gather(b):
        pltpu.make_async_copy(table_hbm.at[idx_v.at[0]], gbufs[b], gsems[b]).wait()

    def start_put(r, b):
        pltpu.async_copy(obufs[b], out_hbm.at[row0 + r, :, pl.ds(0, D_MODEL)], psems[b])

    def wait_put(b):
        pltpu.make_async_copy(obufs[b], out_hbm.at[row0, :, pl.ds(0, D_MODEL)], psems[b]).wait()

    def mul_row(b):
        gb, ob = gbufs[b], obufs[b]

        def body(t):
            for d in range(4):
                sl = pl.ds(d * 16, 16)
                ob[t, sl] = gb[t, sl] * SCALE

        pl.loop(0, SEQ, unroll=4)(body)

    for b in range(NBUF):
        start_gather(b, b)

    for b in range(NBUF):
        wait_gather(b)
        mul_row(b)
        start_put(b, b)
        start_gather(NBUF + b, b)

    def round_body(g):
        for b in range(NBUF):
            r = g * NBUF + b
            wait_gather(b)
            wait_put(b)
            mul_row(b)
            start_put(r, b)
            start_gather(r + NBUF, b)

    pl.loop(1, NROUND - 1)(round_body)

    for b in range(NBUF):
        r = (NROUND - 1) * NBUF + b
        wait_gather(b)
        wait_put(b)
        mul_row(b)
        start_put(r, b)

    for b in range(NBUF):
        wait_put(b)


@jax.jit
def kernel(x, table):
    mesh = plsc.VectorSubcoreMesh(core_axis_name="c", subcore_axis_name="s")
    run = pl.kernel(
        _emb_kernel,
        out_type=jax.ShapeDtypeStruct((BATCH, SEQ, 128), jnp.float32),
        mesh=mesh,
        scratch_types=(
            [pltpu.VMEM((ROWS_PER_W, SEQ), jnp.int32)]
            + [pltpu.VMEM((SEQ, D_MODEL), jnp.float32) for _ in range(2 * NBUF)]
            + [pltpu.SemaphoreType.DMA for _ in range(2 * NBUF)]
        ),
        compiler_params=pltpu.CompilerParams(use_tc_tiling_on_sc=False),
    )
    return run(table + 0.0, x.astype(jnp.int32))[:, :, :D_MODEL]
